# Initial kernel scaffold; baseline (speedup 1.0000x reference)
import jax
import jax.numpy as jnp
from jax.experimental import pallas as pl


def kernel(x, edge, W1, b1, W2, b2):
    raise NotImplementedError("write your pallas kernel here")


# speedup vs baseline: 11.7355x; 11.7355x over previous
import functools

import jax
import jax.numpy as jnp
from jax import lax
from jax.experimental import pallas as pl
from jax.experimental.pallas import tpu as pltpu
from jax.experimental.pallas import tpu_sc as plsc

N = 10000
E = 320000
D_IN = 128
D_OUT = 40
D_OUT_PAD = 48

NC = 2
NS = 16
NW = NC * NS
CH = 128
NCHUNK = 80
EPW = CH * NCHUNK
E_PAD = EPW * NW
N_TAB = 10240
ZROWS = N_TAB // NS

_MESH = dict(core_axis_name="c", subcore_axis_name="s")


def _make_agg(d):
  @functools.partial(
      pl.kernel,
      out_type=jax.ShapeDtypeStruct((NC, N_TAB, d), jnp.float32),
      mesh=plsc.VectorSubcoreMesh(**_MESH),
      compiler_params=pltpu.CompilerParams(use_tc_tiling_on_sc=False),
      scratch_types=[
          pltpu.VMEM((NCHUNK, CH), jnp.int32),
          pltpu.VMEM((NCHUNK, CH), jnp.int32),
          pltpu.VMEM((CH, d), jnp.float32),
          pltpu.VMEM((CH, d), jnp.float32),
          pltpu.SemaphoreType.DMA,
          pltpu.SemaphoreType.DMA,
          pltpu.VMEM_SHARED((N_TAB, d), jnp.float32),
      ],
  )
  def agg(tab_hbm, srcr_hbm, dstr_hbm, zer_hbm, out_hbm,
          src_v, dst_v, buf_a, buf_b, sem_a, sem_b, acc):
    cid = lax.axis_index("c")
    sid = lax.axis_index("s")
    wid = cid * NS + sid
    pltpu.sync_copy(srcr_hbm.at[wid], src_v)
    pltpu.sync_copy(dstr_hbm.at[wid], dst_v)
    pltpu.sync_copy(zer_hbm, acc.at[pl.ds(sid * ZROWS, ZROWS)])
    plsc.subcore_barrier()

    pltpu.async_copy(tab_hbm.at[src_v.at[0]], buf_a, sem_a)

    def body(i, _):
      j = 2 * i
      pltpu.make_async_copy(tab_hbm.at[src_v.at[j]], buf_a, sem_a).wait()
      pltpu.async_copy(tab_hbm.at[src_v.at[j + 1]], buf_b, sem_b)
      pltpu.sync_copy(buf_a, acc.at[dst_v.at[j]], add=True)
      pltpu.make_async_copy(tab_hbm.at[src_v.at[j + 1]], buf_b, sem_b).wait()

      @pl.when(j + 2 < NCHUNK)
      def _():
        pltpu.async_copy(tab_hbm.at[src_v.at[j + 2]], buf_a, sem_a)

      pltpu.sync_copy(buf_b, acc.at[dst_v.at[j + 1]], add=True)
      return 0

    lax.fori_loop(0, NCHUNK // 2, body, 0)
    plsc.subcore_barrier()
    pltpu.sync_copy(acc.at[pl.ds(sid * ZROWS, ZROWS)],
                    out_hbm.at[cid, pl.ds(sid * ZROWS, ZROWS)])

  return agg


@functools.partial(
    pl.kernel,
    out_type=jax.ShapeDtypeStruct((NC, N_TAB, 16), jnp.float32),
    mesh=plsc.VectorSubcoreMesh(**_MESH),
    compiler_params=pltpu.CompilerParams(use_tc_tiling_on_sc=False),
    scratch_types=[
        pltpu.VMEM((NCHUNK, CH), jnp.int32),
        pltpu.VMEM((CH, 16), jnp.float32),
        pltpu.VMEM_SHARED((N_TAB, 16), jnp.float32),
    ],
)
def _deg(dstr_hbm, ones_hbm, zer_hbm, out_hbm, dst_v, ones_v, acc):
  cid = lax.axis_index("c")
  sid = lax.axis_index("s")
  wid = cid * NS + sid
  pltpu.sync_copy(dstr_hbm.at[wid], dst_v)
  pltpu.sync_copy(ones_hbm, ones_v)
  pltpu.sync_copy(zer_hbm, acc.at[pl.ds(sid * ZROWS, ZROWS)])
  plsc.subcore_barrier()

  def body(j, _):
    pltpu.sync_copy(ones_v, acc.at[dst_v.at[j]], add=True)
    return 0

  lax.fori_loop(0, NCHUNK, body, 0)
  plsc.subcore_barrier()
  pltpu.sync_copy(acc.at[pl.ds(sid * ZROWS, ZROWS)],
                  out_hbm.at[cid, pl.ds(sid * ZROWS, ZROWS)])


_R = 1000


def _dinv_of(degp):
  deg = degp[0, :, 0] + degp[1, :, 0] + 1.0
  return lax.rsqrt(deg)


def _mm1_body(degp_ref, x_ref, w_ref, hs_ref):
  dinv = _dinv_of(degp_ref[...])
  h = jnp.dot(x_ref[...], w_ref[...], preferred_element_type=jnp.float32)
  hs_ref[...] = h * dinv[:, None]


def _comb1_body(degp_ref, pl_ref, ph_ref, hs_ref, b1_ref, w2_ref, gs_ref):
  dinv = _dinv_of(degp_ref[...])
  a = pl_ref[...]
  b = ph_ref[...]
  agg = jnp.concatenate([a[0] + a[1], b[0] + b[1]], axis=1)
  s = (agg + hs_ref[...]) * dinv[:, None] + b1_ref[...]
  h1 = jnp.maximum(s, 0.0)
  gs_ref[...] = jnp.dot(h1, w2_ref[...],
                        preferred_element_type=jnp.float32) * dinv[:, None]


def _final_body(degp_ref, q_ref, gs_ref, b2_ref, o_ref):
  dinv = _dinv_of(degp_ref[...])
  q = q_ref[...]
  z = (q[0] + q[1] + gs_ref[...]) * dinv[:, None] + b2_ref[...]
  z = z[:, :D_OUT]
  m = jnp.max(z, axis=1, keepdims=True)
  lse = jnp.log(jnp.sum(jnp.exp(z - m), axis=1, keepdims=True)) + m
  o_ref[...] = z - lse


def _degp_spec():
  return pl.BlockSpec((NC, _R, 16), lambda i: (0, i, 0))


_mm1 = pl.pallas_call(
    _mm1_body,
    grid=(N // _R,),
    in_specs=[
        _degp_spec(),
        pl.BlockSpec((_R, D_IN), lambda i: (i, 0)),
        pl.BlockSpec((D_IN, D_IN), lambda i: (0, 0)),
    ],
    out_specs=pl.BlockSpec((_R, D_IN), lambda i: (i, 0)),
    out_shape=jax.ShapeDtypeStruct((N, D_IN), jnp.float32),
)

_comb1 = pl.pallas_call(
    _comb1_body,
    grid=(N // _R,),
    in_specs=[
        _degp_spec(),
        pl.BlockSpec((NC, _R, D_IN // 2), lambda i: (0, i, 0)),
        pl.BlockSpec((NC, _R, D_IN // 2), lambda i: (0, i, 0)),
        pl.BlockSpec((_R, D_IN), lambda i: (i, 0)),
        pl.BlockSpec((1, D_IN), lambda i: (0, 0)),
        pl.BlockSpec((D_IN, D_OUT_PAD), lambda i: (0, 0)),
    ],
    out_specs=pl.BlockSpec((_R, D_OUT_PAD), lambda i: (i, 0)),
    out_shape=jax.ShapeDtypeStruct((N, D_OUT_PAD), jnp.float32),
)

_final = pl.pallas_call(
    _final_body,
    grid=(N // _R,),
    in_specs=[
        _degp_spec(),
        pl.BlockSpec((NC, _R, D_OUT_PAD), lambda i: (0, i, 0)),
        pl.BlockSpec((_R, D_OUT_PAD), lambda i: (i, 0)),
        pl.BlockSpec((1, D_OUT_PAD), lambda i: (0, 0)),
    ],
    out_specs=pl.BlockSpec((_R, D_OUT), lambda i: (i, 0)),
    out_shape=jax.ShapeDtypeStruct((N, D_OUT), jnp.float32),
)

_agg64 = _make_agg(D_IN // 2)
_agg48 = _make_agg(D_OUT_PAD)


def kernel(x, edge, W1, b1, W2, b2):
  pad = E_PAD - E
  src = jnp.concatenate([edge[0], jnp.zeros((pad,), jnp.int32)])
  dst = jnp.concatenate([edge[1], jnp.full((pad,), N, jnp.int32)])
  srcr = src.reshape(NW, NCHUNK, CH)
  dstr = dst.reshape(NW, NCHUNK, CH)
  zer64 = jnp.zeros((ZROWS, D_IN // 2), jnp.float32)
  zer48 = jnp.zeros((ZROWS, D_OUT_PAD), jnp.float32)
  zer16 = jnp.zeros((ZROWS, 16), jnp.float32)
  ones16 = jnp.ones((CH, 16), jnp.float32)
  w2p = jnp.pad(W2, ((0, 0), (0, D_OUT_PAD - D_OUT)))
  b1r = b1.reshape(1, D_IN)
  b2r = jnp.pad(b2, (0, D_OUT_PAD - D_OUT)).reshape(1, D_OUT_PAD)

  degp = _deg(dstr, ones16, zer16)
  hs = _mm1(degp, x, W1)
  hsl = lax.slice(hs, (0, 0), (N, D_IN // 2))
  hsh = lax.slice(hs, (0, D_IN // 2), (N, D_IN))
  p1l = _agg64(hsl, srcr, dstr, zer64)
  p1h = _agg64(hsh, srcr, dstr, zer64)
  gs = _comb1(degp, p1l, p1h, hs, b1r, w2p)
  p2 = _agg48(gs, srcr, dstr, zer48)
  return _final(degp, p2, gs, b2r)


# speedup vs baseline: 15.6278x; 1.3317x over previous
import functools

import jax
import jax.numpy as jnp
from jax import lax
from jax.experimental import pallas as pl
from jax.experimental.pallas import tpu as pltpu
from jax.experimental.pallas import tpu_sc as plsc

N = 10000
E = 320000
D_IN = 128
D_OUT = 40
D_OUT_PAD = 48

NC = 2
NS = 16
NW = NC * NS
CH = 128
NCHUNK = 80
EPW = CH * NCHUNK
E_PAD = EPW * NW
N_TAB = 10240
ZROWS = N_TAB // NS

_MESH = dict(core_axis_name="c", subcore_axis_name="s")


def _make_agg(d):
  @functools.partial(
      pl.kernel,
      out_type=jax.ShapeDtypeStruct((NC, N_TAB, d), jnp.float32),
      mesh=plsc.VectorSubcoreMesh(**_MESH),
      compiler_params=pltpu.CompilerParams(use_tc_tiling_on_sc=False),
      scratch_types=[
          pltpu.VMEM((NCHUNK, CH), jnp.int32),
          pltpu.VMEM((NCHUNK, CH), jnp.int32),
          pltpu.VMEM((CH, d), jnp.float32),
          pltpu.VMEM((CH, d), jnp.float32),
          pltpu.SemaphoreType.DMA,
          pltpu.SemaphoreType.DMA,
          pltpu.VMEM_SHARED((N_TAB, d), jnp.float32),
      ],
  )
  def agg(tab_hbm, srcr_hbm, dstr_hbm, zer_hbm, out_hbm,
          src_v, dst_v, buf_a, buf_b, sem_a, sem_b, acc):
    cid = lax.axis_index("c")
    sid = lax.axis_index("s")
    wid = cid * NS + sid
    pltpu.sync_copy(srcr_hbm.at[wid], src_v)
    pltpu.sync_copy(dstr_hbm.at[wid], dst_v)
    pltpu.sync_copy(zer_hbm, acc.at[pl.ds(sid * ZROWS, ZROWS)])
    plsc.subcore_barrier()

    pltpu.async_copy(tab_hbm.at[src_v.at[0]], buf_a, sem_a)

    def body(i, _):
      j = 2 * i
      pltpu.make_async_copy(tab_hbm.at[src_v.at[j]], buf_a, sem_a).wait()
      pltpu.async_copy(tab_hbm.at[src_v.at[j + 1]], buf_b, sem_b)
      pltpu.sync_copy(buf_a, acc.at[dst_v.at[j]], add=True)
      pltpu.make_async_copy(tab_hbm.at[src_v.at[j + 1]], buf_b, sem_b).wait()

      @pl.when(j + 2 < NCHUNK)
      def _():
        pltpu.async_copy(tab_hbm.at[src_v.at[j + 2]], buf_a, sem_a)

      pltpu.sync_copy(buf_b, acc.at[dst_v.at[j + 1]], add=True)
      return 0

    lax.fori_loop(0, NCHUNK // 2, body, 0)
    plsc.subcore_barrier()
    pltpu.sync_copy(acc.at[pl.ds(sid * ZROWS, ZROWS)],
                    out_hbm.at[cid, pl.ds(sid * ZROWS, ZROWS)])

  return agg


NCHUNK2 = NCHUNK * NC
DH = D_IN // 2


@functools.partial(
    pl.kernel,
    out_type=jax.ShapeDtypeStruct((NC, N_TAB, DH), jnp.float32),
    mesh=plsc.VectorSubcoreMesh(**_MESH),
    compiler_params=pltpu.CompilerParams(use_tc_tiling_on_sc=False),
    scratch_types=[
        pltpu.VMEM((NCHUNK2, CH), jnp.int32),
        pltpu.VMEM((NCHUNK2, CH), jnp.int32),
        pltpu.VMEM((CH, DH), jnp.float32),
        pltpu.VMEM((CH, DH), jnp.float32),
        pltpu.SemaphoreType.DMA,
        pltpu.SemaphoreType.DMA,
        pltpu.VMEM_SHARED((N_TAB, DH), jnp.float32),
    ],
)
def _agg_split(tab_hbm, srcr_hbm, dstr_hbm, zer_hbm, out_hbm,
               src_v, dst_v, buf_a, buf_b, sem_a, sem_b, acc):
  cid = lax.axis_index("c")
  sid = lax.axis_index("s")
  tab = tab_hbm.at[cid]
  pltpu.sync_copy(srcr_hbm.at[sid], src_v)
  pltpu.sync_copy(dstr_hbm.at[sid], dst_v)
  pltpu.sync_copy(zer_hbm, acc.at[pl.ds(sid * ZROWS, ZROWS)])
  plsc.subcore_barrier()

  pltpu.async_copy(tab.at[src_v.at[0]], buf_a, sem_a)

  def body(i, _):
    j = 2 * i
    pltpu.make_async_copy(tab.at[src_v.at[j]], buf_a, sem_a).wait()
    pltpu.async_copy(tab.at[src_v.at[j + 1]], buf_b, sem_b)
    pltpu.sync_copy(buf_a, acc.at[dst_v.at[j]], add=True)
    pltpu.make_async_copy(tab.at[src_v.at[j + 1]], buf_b, sem_b).wait()

    @pl.when(j + 2 < NCHUNK2)
    def _():
      pltpu.async_copy(tab.at[src_v.at[j + 2]], buf_a, sem_a)

    pltpu.sync_copy(buf_b, acc.at[dst_v.at[j + 1]], add=True)
    return 0

  lax.fori_loop(0, NCHUNK2 // 2, body, 0)
  plsc.subcore_barrier()
  pltpu.sync_copy(acc.at[pl.ds(sid * ZROWS, ZROWS)],
                  out_hbm.at[cid, pl.ds(sid * ZROWS, ZROWS)])


@functools.partial(
    pl.kernel,
    out_type=jax.ShapeDtypeStruct((NC, N_TAB, 16), jnp.float32),
    mesh=plsc.VectorSubcoreMesh(**_MESH),
    compiler_params=pltpu.CompilerParams(use_tc_tiling_on_sc=False),
    scratch_types=[
        pltpu.VMEM((NCHUNK, CH), jnp.int32),
        pltpu.VMEM((CH, 16), jnp.float32),
        pltpu.VMEM_SHARED((N_TAB, 16), jnp.float32),
    ],
)
def _deg(dstr_hbm, ones_hbm, zer_hbm, out_hbm, dst_v, ones_v, acc):
  cid = lax.axis_index("c")
  sid = lax.axis_index("s")
  wid = cid * NS + sid
  pltpu.sync_copy(dstr_hbm.at[wid], dst_v)
  pltpu.sync_copy(ones_hbm, ones_v)
  pltpu.sync_copy(zer_hbm, acc.at[pl.ds(sid * ZROWS, ZROWS)])
  plsc.subcore_barrier()

  def body(j, _):
    pltpu.sync_copy(ones_v, acc.at[dst_v.at[j]], add=True)
    return 0

  lax.fori_loop(0, NCHUNK, body, 0)
  plsc.subcore_barrier()
  pltpu.sync_copy(acc.at[pl.ds(sid * ZROWS, ZROWS)],
                  out_hbm.at[cid, pl.ds(sid * ZROWS, ZROWS)])


_R = 1000


def _dinv_of(degp):
  deg = degp[0, :, 0] + degp[1, :, 0] + 1.0
  return lax.rsqrt(deg)


def _mm1_body(degp_ref, x_ref, w_ref, hs_ref):
  dinv = _dinv_of(degp_ref[...])
  h = jnp.dot(x_ref[...], w_ref[...], preferred_element_type=jnp.float32)
  hs = h * dinv[:, None]
  hs_ref[...] = jnp.stack([hs[:, :DH], hs[:, DH:]])


def _comb1_body(degp_ref, p_ref, hs_ref, b1_ref, w2_ref, gs_ref):
  dinv = _dinv_of(degp_ref[...])
  p = p_ref[...]
  hsp = hs_ref[...]
  agg = jnp.concatenate([p[0] + hsp[0], p[1] + hsp[1]], axis=1)
  s = agg * dinv[:, None] + b1_ref[...]
  h1 = jnp.maximum(s, 0.0)
  gs_ref[...] = jnp.dot(h1, w2_ref[...],
                        preferred_element_type=jnp.float32) * dinv[:, None]


def _final_body(degp_ref, q_ref, gs_ref, b2_ref, o_ref):
  dinv = _dinv_of(degp_ref[...])
  q = q_ref[...]
  z = (q[0] + q[1] + gs_ref[...]) * dinv[:, None] + b2_ref[...]
  z = z[:, :D_OUT]
  m = jnp.max(z, axis=1, keepdims=True)
  lse = jnp.log(jnp.sum(jnp.exp(z - m), axis=1, keepdims=True)) + m
  o_ref[...] = z - lse


def _degp_spec():
  return pl.BlockSpec((NC, _R, 16), lambda i: (0, i, 0))


_mm1 = pl.pallas_call(
    _mm1_body,
    grid=(N // _R,),
    in_specs=[
        _degp_spec(),
        pl.BlockSpec((_R, D_IN), lambda i: (i, 0)),
        pl.BlockSpec((D_IN, D_IN), lambda i: (0, 0)),
    ],
    out_specs=pl.BlockSpec((NC, _R, DH), lambda i: (0, i, 0)),
    out_shape=jax.ShapeDtypeStruct((NC, N, DH), jnp.float32),
)

_comb1 = pl.pallas_call(
    _comb1_body,
    grid=(N // _R,),
    in_specs=[
        _degp_spec(),
        pl.BlockSpec((NC, _R, DH), lambda i: (0, i, 0)),
        pl.BlockSpec((NC, _R, DH), lambda i: (0, i, 0)),
        pl.BlockSpec((1, D_IN), lambda i: (0, 0)),
        pl.BlockSpec((D_IN, D_OUT_PAD), lambda i: (0, 0)),
    ],
    out_specs=pl.BlockSpec((_R, D_OUT_PAD), lambda i: (i, 0)),
    out_shape=jax.ShapeDtypeStruct((N, D_OUT_PAD), jnp.float32),
)

_final = pl.pallas_call(
    _final_body,
    grid=(N // _R,),
    in_specs=[
        _degp_spec(),
        pl.BlockSpec((NC, _R, D_OUT_PAD), lambda i: (0, i, 0)),
        pl.BlockSpec((_R, D_OUT_PAD), lambda i: (i, 0)),
        pl.BlockSpec((1, D_OUT_PAD), lambda i: (0, 0)),
    ],
    out_specs=pl.BlockSpec((_R, D_OUT), lambda i: (i, 0)),
    out_shape=jax.ShapeDtypeStruct((N, D_OUT), jnp.float32),
)

_agg48 = _make_agg(D_OUT_PAD)


def kernel(x, edge, W1, b1, W2, b2):
  pad = E_PAD - E
  src = jnp.concatenate([edge[0], jnp.zeros((pad,), jnp.int32)])
  dst = jnp.concatenate([edge[1], jnp.full((pad,), N, jnp.int32)])
  srcr = src.reshape(NW, NCHUNK, CH)
  dstr = dst.reshape(NW, NCHUNK, CH)
  srcr2 = src.reshape(NS, NCHUNK2, CH)
  dstr2 = dst.reshape(NS, NCHUNK2, CH)
  zer64 = jnp.zeros((ZROWS, DH), jnp.float32)
  zer48 = jnp.zeros((ZROWS, D_OUT_PAD), jnp.float32)
  zer16 = jnp.zeros((ZROWS, 16), jnp.float32)
  ones16 = jnp.ones((CH, 16), jnp.float32)
  w2p = jnp.pad(W2, ((0, 0), (0, D_OUT_PAD - D_OUT)))
  b1r = b1.reshape(1, D_IN)
  b2r = jnp.pad(b2, (0, D_OUT_PAD - D_OUT)).reshape(1, D_OUT_PAD)

  degp = _deg(dstr, ones16, zer16)
  hsp = _mm1(degp, x, W1)
  p1 = _agg_split(hsp, srcr2, dstr2, zer64)
  gs = _comb1(degp, p1, hsp, b1r, w2p)
  p2 = _agg48(gs, srcr, dstr, zer48)
  return _final(degp, p2, gs, b2r)


# speedup vs baseline: 17.7437x; 1.1354x over previous
import functools

import jax
import jax.numpy as jnp
from jax import lax
from jax.experimental import pallas as pl
from jax.experimental.pallas import tpu as pltpu
from jax.experimental.pallas import tpu_sc as plsc

N = 10000
E = 320000
D_IN = 128
D_OUT = 40
D_OUT_PAD = 48

NC = 2
NS = 16
NW = NC * NS
CH = 128
NCHUNK = 80
EPW = CH * NCHUNK
E_PAD = EPW * NW
N_TAB = 10240
ZROWS = N_TAB // NS

_MESH = dict(core_axis_name="c", subcore_axis_name="s")
_NBUF = 4


def _edge_pipeline(tab, src_v, dst_v, acc, bufs, gsems, ssems, nchunk):
  for k in range(_NBUF - 1):
    pltpu.async_copy(tab.at[src_v.at[k]], bufs[k], gsems[k])

  def body(i, _):
    for k in range(_NBUF):
      j = _NBUF * i + k
      kn = (k + _NBUF - 1) % _NBUF
      pltpu.make_async_copy(tab.at[src_v.at[j]], bufs[k], gsems[k]).wait()
      pltpu.async_copy(bufs[k], acc.at[dst_v.at[j]], ssems[k], add=True)

      @pl.when(j >= 1)
      def _():
        pltpu.make_async_copy(bufs[kn], acc.at[dst_v.at[j - 1]],
                              ssems[kn]).wait()

      @pl.when(j + _NBUF - 1 < nchunk)
      def _():
        pltpu.async_copy(tab.at[src_v.at[j + _NBUF - 1]], bufs[kn], gsems[kn])

    return 0

  lax.fori_loop(0, nchunk // _NBUF, body, 0)
  pltpu.make_async_copy(bufs[_NBUF - 1], acc.at[dst_v.at[nchunk - 1]],
                        ssems[_NBUF - 1]).wait()


def _make_agg(d):
  @functools.partial(
      pl.kernel,
      out_type=jax.ShapeDtypeStruct((NC, N_TAB, d), jnp.float32),
      mesh=plsc.VectorSubcoreMesh(**_MESH),
      compiler_params=pltpu.CompilerParams(use_tc_tiling_on_sc=False),
      scratch_types=(
          [pltpu.VMEM((NCHUNK, CH), jnp.int32)] * 2
          + [pltpu.VMEM((CH, d), jnp.float32)] * _NBUF
          + [pltpu.SemaphoreType.DMA] * (2 * _NBUF)
          + [pltpu.VMEM_SHARED((N_TAB, d), jnp.float32)]
      ),
  )
  def agg(tab_hbm, srcr_hbm, dstr_hbm, zer_hbm, out_hbm,
          src_v, dst_v, *rest):
    bufs, sems, acc = rest[:_NBUF], rest[_NBUF:3 * _NBUF], rest[-1]
    gsems, ssems = sems[:_NBUF], sems[_NBUF:]
    cid = lax.axis_index("c")
    sid = lax.axis_index("s")
    wid = cid * NS + sid
    pltpu.sync_copy(srcr_hbm.at[wid], src_v)
    pltpu.sync_copy(dstr_hbm.at[wid], dst_v)
    pltpu.sync_copy(zer_hbm, acc.at[pl.ds(sid * ZROWS, ZROWS)])
    plsc.subcore_barrier()
    _edge_pipeline(tab_hbm, src_v, dst_v, acc, bufs, gsems, ssems, NCHUNK)
    plsc.subcore_barrier()
    pltpu.sync_copy(acc.at[pl.ds(sid * ZROWS, ZROWS)],
                    out_hbm.at[cid, pl.ds(sid * ZROWS, ZROWS)])

  return agg


NCHUNK2 = NCHUNK * NC
DH = D_IN // 2


@functools.partial(
    pl.kernel,
    out_type=jax.ShapeDtypeStruct((NC, N_TAB, DH), jnp.float32),
    mesh=plsc.VectorSubcoreMesh(**_MESH),
    compiler_params=pltpu.CompilerParams(use_tc_tiling_on_sc=False),
    scratch_types=(
        [pltpu.VMEM((NCHUNK2, CH), jnp.int32)] * 2
        + [pltpu.VMEM((CH, DH), jnp.float32)] * _NBUF
        + [pltpu.SemaphoreType.DMA] * (2 * _NBUF)
        + [pltpu.VMEM_SHARED((N_TAB, DH), jnp.float32)]
    ),
)
def _agg_split(tab_hbm, srcr_hbm, dstr_hbm, zer_hbm, out_hbm,
               src_v, dst_v, *rest):
  bufs, sems, acc = rest[:_NBUF], rest[_NBUF:3 * _NBUF], rest[-1]
  gsems, ssems = sems[:_NBUF], sems[_NBUF:]
  cid = lax.axis_index("c")
  sid = lax.axis_index("s")
  tab = tab_hbm.at[cid]
  pltpu.sync_copy(srcr_hbm.at[sid], src_v)
  pltpu.sync_copy(dstr_hbm.at[sid], dst_v)
  pltpu.sync_copy(zer_hbm, acc.at[pl.ds(sid * ZROWS, ZROWS)])
  plsc.subcore_barrier()
  _edge_pipeline(tab, src_v, dst_v, acc, bufs, gsems, ssems, NCHUNK2)
  plsc.subcore_barrier()
  pltpu.sync_copy(acc.at[pl.ds(sid * ZROWS, ZROWS)],
                  out_hbm.at[cid, pl.ds(sid * ZROWS, ZROWS)])


@functools.partial(
    pl.kernel,
    out_type=jax.ShapeDtypeStruct((NC, N_TAB, 16), jnp.float32),
    mesh=plsc.VectorSubcoreMesh(**_MESH),
    compiler_params=pltpu.CompilerParams(use_tc_tiling_on_sc=False),
    scratch_types=[
        pltpu.VMEM((NCHUNK, CH), jnp.int32),
        pltpu.VMEM((CH, 16), jnp.float32),
        pltpu.VMEM_SHARED((N_TAB, 16), jnp.float32),
    ],
)
def _deg(dstr_hbm, ones_hbm, zer_hbm, out_hbm, dst_v, ones_v, acc):
  cid = lax.axis_index("c")
  sid = lax.axis_index("s")
  wid = cid * NS + sid
  pltpu.sync_copy(dstr_hbm.at[wid], dst_v)
  pltpu.sync_copy(ones_hbm, ones_v)
  pltpu.sync_copy(zer_hbm, acc.at[pl.ds(sid * ZROWS, ZROWS)])
  plsc.subcore_barrier()

  def body(j, _):
    pltpu.sync_copy(ones_v, acc.at[dst_v.at[j]], add=True)
    return 0

  lax.fori_loop(0, NCHUNK, body, 0)
  plsc.subcore_barrier()
  pltpu.sync_copy(acc.at[pl.ds(sid * ZROWS, ZROWS)],
                  out_hbm.at[cid, pl.ds(sid * ZROWS, ZROWS)])


_R = 1000


def _dinv_of(degp):
  deg = degp[0, :, 0] + degp[1, :, 0] + 1.0
  return lax.rsqrt(deg)


def _mm1_body(degp_ref, x_ref, w_ref, hs_ref):
  dinv = _dinv_of(degp_ref[...])
  h = jnp.dot(x_ref[...], w_ref[...], preferred_element_type=jnp.float32)
  hs = h * dinv[:, None]
  hs_ref[...] = jnp.stack([hs[:, :DH], hs[:, DH:]])


def _comb1_body(degp_ref, p_ref, hs_ref, b1_ref, w2_ref, gs_ref):
  dinv = _dinv_of(degp_ref[...])
  p = p_ref[...]
  hsp = hs_ref[...]
  agg = jnp.concatenate([p[0] + hsp[0], p[1] + hsp[1]], axis=1)
  s = agg * dinv[:, None] + b1_ref[...]
  h1 = jnp.maximum(s, 0.0)
  gs_ref[...] = jnp.dot(h1, w2_ref[...],
                        preferred_element_type=jnp.float32) * dinv[:, None]


def _final_body(degp_ref, q_ref, gs_ref, b2_ref, o_ref):
  dinv = _dinv_of(degp_ref[...])
  q = q_ref[...]
  z = (q[0] + q[1] + gs_ref[...]) * dinv[:, None] + b2_ref[...]
  z = z[:, :D_OUT]
  m = jnp.max(z, axis=1, keepdims=True)
  lse = jnp.log(jnp.sum(jnp.exp(z - m), axis=1, keepdims=True)) + m
  o_ref[...] = z - lse


def _degp_spec():
  return pl.BlockSpec((NC, _R, 16), lambda i: (0, i, 0))


_mm1 = pl.pallas_call(
    _mm1_body,
    grid=(N // _R,),
    in_specs=[
        _degp_spec(),
        pl.BlockSpec((_R, D_IN), lambda i: (i, 0)),
        pl.BlockSpec((D_IN, D_IN), lambda i: (0, 0)),
    ],
    out_specs=pl.BlockSpec((NC, _R, DH), lambda i: (0, i, 0)),
    out_shape=jax.ShapeDtypeStruct((NC, N, DH), jnp.float32),
)

_comb1 = pl.pallas_call(
    _comb1_body,
    grid=(N // _R,),
    in_specs=[
        _degp_spec(),
        pl.BlockSpec((NC, _R, DH), lambda i: (0, i, 0)),
        pl.BlockSpec((NC, _R, DH), lambda i: (0, i, 0)),
        pl.BlockSpec((1, D_IN), lambda i: (0, 0)),
        pl.BlockSpec((D_IN, D_OUT_PAD), lambda i: (0, 0)),
    ],
    out_specs=pl.BlockSpec((_R, D_OUT_PAD), lambda i: (i, 0)),
    out_shape=jax.ShapeDtypeStruct((N, D_OUT_PAD), jnp.float32),
)

_final = pl.pallas_call(
    _final_body,
    grid=(N // _R,),
    in_specs=[
        _degp_spec(),
        pl.BlockSpec((NC, _R, D_OUT_PAD), lambda i: (0, i, 0)),
        pl.BlockSpec((_R, D_OUT_PAD), lambda i: (i, 0)),
        pl.BlockSpec((1, D_OUT_PAD), lambda i: (0, 0)),
    ],
    out_specs=pl.BlockSpec((_R, D_OUT), lambda i: (i, 0)),
    out_shape=jax.ShapeDtypeStruct((N, D_OUT), jnp.float32),
)

_agg48 = _make_agg(D_OUT_PAD)


def kernel(x, edge, W1, b1, W2, b2):
  pad = E_PAD - E
  src = jnp.concatenate([edge[0], jnp.zeros((pad,), jnp.int32)])
  dst = jnp.concatenate([edge[1], jnp.full((pad,), N, jnp.int32)])
  srcr = src.reshape(NW, NCHUNK, CH)
  dstr = dst.reshape(NW, NCHUNK, CH)
  srcr2 = src.reshape(NS, NCHUNK2, CH)
  dstr2 = dst.reshape(NS, NCHUNK2, CH)
  zer64 = jnp.zeros((ZROWS, DH), jnp.float32)
  zer48 = jnp.zeros((ZROWS, D_OUT_PAD), jnp.float32)
  zer16 = jnp.zeros((ZROWS, 16), jnp.float32)
  ones16 = jnp.ones((CH, 16), jnp.float32)
  w2p = jnp.pad(W2, ((0, 0), (0, D_OUT_PAD - D_OUT)))
  b1r = b1.reshape(1, D_IN)
  b2r = jnp.pad(b2, (0, D_OUT_PAD - D_OUT)).reshape(1, D_OUT_PAD)

  degp = _deg(dstr, ones16, zer16)
  hsp = _mm1(degp, x, W1)
  p1 = _agg_split(hsp, srcr2, dstr2, zer64)
  gs = _comb1(degp, p1, hsp, b1r, w2p)
  p2 = _agg48(gs, srcr, dstr, zer48)
  return _final(degp, p2, gs, b2r)


# speedup vs baseline: 35.1555x; 1.9813x over previous
import functools

import jax
import jax.numpy as jnp
from jax import lax
from jax.experimental import pallas as pl
from jax.experimental.pallas import tpu as pltpu
from jax.experimental.pallas import tpu_sc as plsc

N = 10000
E = 320000
D_IN = 128
D_OUT = 40
D_OUT_PAD = 48

NC = 2
NS = 16
NW = NC * NS
CH = 128
NCHUNK = 80
EPW = CH * NCHUNK
E_PAD = EPW * NW
N_TAB = 10240
ZROWS = N_TAB // NS

_MESH = dict(core_axis_name="c", subcore_axis_name="s")
_NBUF = 4


def _edge_pipeline(tab, src_v, dst_v, acc, bufs, gsems, ssems, nchunk):
  for k in range(_NBUF - 1):
    pltpu.async_copy(tab.at[src_v.at[k]], bufs[k], gsems[k])

  def body(i, _):
    for k in range(_NBUF):
      j = _NBUF * i + k
      kn = (k + _NBUF - 1) % _NBUF
      pltpu.make_async_copy(tab.at[src_v.at[j]], bufs[k], gsems[k]).wait()
      pltpu.async_copy(bufs[k], acc.at[dst_v.at[j]], ssems[k], add=True)

      @pl.when(j >= 1)
      def _():
        pltpu.make_async_copy(bufs[kn], acc.at[dst_v.at[j - 1]],
                              ssems[kn]).wait()

      @pl.when(j + _NBUF - 1 < nchunk)
      def _():
        pltpu.async_copy(tab.at[src_v.at[j + _NBUF - 1]], bufs[kn], gsems[kn])

    return 0

  lax.fori_loop(0, nchunk // _NBUF, body, 0)
  pltpu.make_async_copy(bufs[_NBUF - 1], acc.at[dst_v.at[nchunk - 1]],
                        ssems[_NBUF - 1]).wait()


def _make_agg(d):
  @functools.partial(
      pl.kernel,
      out_type=jax.ShapeDtypeStruct((NC, N_TAB, d), jnp.float32),
      mesh=plsc.VectorSubcoreMesh(**_MESH),
      compiler_params=pltpu.CompilerParams(use_tc_tiling_on_sc=False),
      scratch_types=(
          [pltpu.VMEM((NCHUNK, CH), jnp.int32)] * 2
          + [pltpu.VMEM((CH, d), jnp.float32)] * _NBUF
          + [pltpu.SemaphoreType.DMA] * (2 * _NBUF)
          + [pltpu.VMEM_SHARED((N_TAB, d), jnp.float32)]
      ),
  )
  def agg(tab_hbm, srcr_hbm, dstr_hbm, zer_hbm, out_hbm,
          src_v, dst_v, *rest):
    bufs, sems, acc = rest[:_NBUF], rest[_NBUF:3 * _NBUF], rest[-1]
    gsems, ssems = sems[:_NBUF], sems[_NBUF:]
    cid = lax.axis_index("c")
    sid = lax.axis_index("s")
    pltpu.sync_copy(srcr_hbm.at[sid, pl.ds(cid * NCHUNK, NCHUNK)], src_v)
    pltpu.sync_copy(dstr_hbm.at[sid, pl.ds(cid * NCHUNK, NCHUNK)], dst_v)
    pltpu.sync_copy(zer_hbm, acc.at[pl.ds(sid * ZROWS, ZROWS)])
    plsc.subcore_barrier()
    _edge_pipeline(tab_hbm, src_v, dst_v, acc, bufs, gsems, ssems, NCHUNK)
    plsc.subcore_barrier()
    pltpu.sync_copy(acc.at[pl.ds(sid * ZROWS, ZROWS)],
                    out_hbm.at[cid, pl.ds(sid * ZROWS, ZROWS)])

  return agg


NCHUNK2 = NCHUNK * NC
DH = D_IN // 2


@functools.partial(
    pl.kernel,
    out_type=jax.ShapeDtypeStruct((NC, N_TAB, DH), jnp.float32),
    mesh=plsc.VectorSubcoreMesh(**_MESH),
    compiler_params=pltpu.CompilerParams(use_tc_tiling_on_sc=False),
    scratch_types=(
        [pltpu.VMEM((NCHUNK2, CH), jnp.int32)] * 2
        + [pltpu.VMEM((CH, DH), jnp.float32)] * _NBUF
        + [pltpu.SemaphoreType.DMA] * (2 * _NBUF)
        + [pltpu.VMEM_SHARED((N_TAB, DH), jnp.float32)]
    ),
)
def _agg_split(tab_hbm, srcr_hbm, dstr_hbm, zer_hbm, out_hbm,
               src_v, dst_v, *rest):
  bufs, sems, acc = rest[:_NBUF], rest[_NBUF:3 * _NBUF], rest[-1]
  gsems, ssems = sems[:_NBUF], sems[_NBUF:]
  cid = lax.axis_index("c")
  sid = lax.axis_index("s")
  tab = tab_hbm.at[cid]
  pltpu.sync_copy(srcr_hbm.at[sid], src_v)
  pltpu.sync_copy(dstr_hbm.at[sid], dst_v)
  pltpu.sync_copy(zer_hbm, acc.at[pl.ds(sid * ZROWS, ZROWS)])
  plsc.subcore_barrier()
  _edge_pipeline(tab, src_v, dst_v, acc, bufs, gsems, ssems, NCHUNK2)
  plsc.subcore_barrier()
  pltpu.sync_copy(acc.at[pl.ds(sid * ZROWS, ZROWS)],
                  out_hbm.at[cid, pl.ds(sid * ZROWS, ZROWS)])


@functools.partial(
    pl.kernel,
    out_type=jax.ShapeDtypeStruct((NC, N_TAB, 16), jnp.float32),
    mesh=plsc.VectorSubcoreMesh(**_MESH),
    compiler_params=pltpu.CompilerParams(use_tc_tiling_on_sc=False),
    scratch_types=[
        pltpu.VMEM((NCHUNK, CH), jnp.int32),
        pltpu.VMEM((CH, 16), jnp.float32),
        pltpu.VMEM_SHARED((N_TAB, 16), jnp.float32),
    ],
)
def _deg(dstr_hbm, ones_hbm, zer_hbm, out_hbm, dst_v, ones_v, acc):
  cid = lax.axis_index("c")
  sid = lax.axis_index("s")
  pltpu.sync_copy(dstr_hbm.at[sid, pl.ds(cid * NCHUNK, NCHUNK)], dst_v)
  pltpu.sync_copy(ones_hbm, ones_v)
  pltpu.sync_copy(zer_hbm, acc.at[pl.ds(sid * ZROWS, ZROWS)])
  plsc.subcore_barrier()

  def body(j, _):
    pltpu.sync_copy(ones_v, acc.at[dst_v.at[j]], add=True)
    return 0

  lax.fori_loop(0, NCHUNK, body, 0)
  plsc.subcore_barrier()
  pltpu.sync_copy(acc.at[pl.ds(sid * ZROWS, ZROWS)],
                  out_hbm.at[cid, pl.ds(sid * ZROWS, ZROWS)])


_R = 1000


def _dinv_of(degp):
  deg = degp[0, :, 0] + degp[1, :, 0] + 1.0
  return lax.rsqrt(deg)


def _mm1_body(degp_ref, x_ref, w_ref, hs_ref):
  dinv = _dinv_of(degp_ref[...])
  h = jnp.dot(x_ref[...], w_ref[...], preferred_element_type=jnp.float32)
  hs = h * dinv[:, None]
  hs_ref[...] = jnp.stack([hs[:, :DH], hs[:, DH:]])


def _comb1_body(degp_ref, p_ref, hs_ref, b1_ref, w2_ref, gs_ref):
  dinv = _dinv_of(degp_ref[...])
  p = p_ref[...]
  hsp = hs_ref[...]
  agg = jnp.concatenate([p[0] + hsp[0], p[1] + hsp[1]], axis=1)
  s = agg * dinv[:, None] + b1_ref[...]
  h1 = jnp.maximum(s, 0.0)
  gs_ref[...] = jnp.dot(h1, w2_ref[...],
                        preferred_element_type=jnp.float32) * dinv[:, None]


def _final_body(degp_ref, q_ref, gs_ref, b2_ref, o_ref):
  dinv = _dinv_of(degp_ref[...])
  q = q_ref[...]
  z = (q[0] + q[1] + gs_ref[...]) * dinv[:, None] + b2_ref[...]
  z = z[:, :D_OUT]
  m = jnp.max(z, axis=1, keepdims=True)
  lse = jnp.log(jnp.sum(jnp.exp(z - m), axis=1, keepdims=True)) + m
  o_ref[...] = z - lse


def _degp_spec():
  return pl.BlockSpec((NC, _R, 16), lambda i: (0, i, 0))


_mm1 = pl.pallas_call(
    _mm1_body,
    grid=(N // _R,),
    in_specs=[
        _degp_spec(),
        pl.BlockSpec((_R, D_IN), lambda i: (i, 0)),
        pl.BlockSpec((D_IN, D_IN), lambda i: (0, 0)),
    ],
    out_specs=pl.BlockSpec((NC, _R, DH), lambda i: (0, i, 0)),
    out_shape=jax.ShapeDtypeStruct((NC, N, DH), jnp.float32),
)

_comb1 = pl.pallas_call(
    _comb1_body,
    grid=(N // _R,),
    in_specs=[
        _degp_spec(),
        pl.BlockSpec((NC, _R, DH), lambda i: (0, i, 0)),
        pl.BlockSpec((NC, _R, DH), lambda i: (0, i, 0)),
        pl.BlockSpec((1, D_IN), lambda i: (0, 0)),
        pl.BlockSpec((D_IN, D_OUT_PAD), lambda i: (0, 0)),
    ],
    out_specs=pl.BlockSpec((_R, D_OUT_PAD), lambda i: (i, 0)),
    out_shape=jax.ShapeDtypeStruct((N, D_OUT_PAD), jnp.float32),
)

_final = pl.pallas_call(
    _final_body,
    grid=(N // _R,),
    in_specs=[
        _degp_spec(),
        pl.BlockSpec((NC, _R, D_OUT_PAD), lambda i: (0, i, 0)),
        pl.BlockSpec((_R, D_OUT_PAD), lambda i: (i, 0)),
        pl.BlockSpec((1, D_OUT_PAD), lambda i: (0, 0)),
    ],
    out_specs=pl.BlockSpec((_R, D_OUT), lambda i: (i, 0)),
    out_shape=jax.ShapeDtypeStruct((N, D_OUT), jnp.float32),
)

_agg48 = _make_agg(D_OUT_PAD)


def kernel(x, edge, W1, b1, W2, b2):
  pad = E_PAD - E
  padi = jnp.arange(pad, dtype=jnp.int32)
  src = jnp.concatenate([edge[0], padi % N])
  dst = jnp.concatenate([edge[1], N + padi % (N_TAB - N)])
  srcr2 = src.reshape(NS, NCHUNK2, CH)
  dstr2 = dst.reshape(NS, NCHUNK2, CH)
  zer64 = jnp.zeros((ZROWS, DH), jnp.float32)
  zer48 = jnp.zeros((ZROWS, D_OUT_PAD), jnp.float32)
  zer16 = jnp.zeros((ZROWS, 16), jnp.float32)
  ones16 = jnp.ones((CH, 16), jnp.float32)
  w2p = jnp.pad(W2, ((0, 0), (0, D_OUT_PAD - D_OUT)))
  b1r = b1.reshape(1, D_IN)
  b2r = jnp.pad(b2, (0, D_OUT_PAD - D_OUT)).reshape(1, D_OUT_PAD)

  degp = _deg(dstr2, ones16, zer16)
  hsp = _mm1(degp, x, W1)
  p1 = _agg_split(hsp, srcr2, dstr2, zer64)
  gs = _comb1(degp, p1, hsp, b1r, w2p)
  p2 = _agg48(gs, srcr2, dstr2, zer48)
  return _final(degp, p2, gs, b2r)


# speedup vs baseline: 35.4104x; 1.0073x over previous
import functools

import jax
import jax.numpy as jnp
from jax import lax
from jax.experimental import pallas as pl
from jax.experimental.pallas import tpu as pltpu
from jax.experimental.pallas import tpu_sc as plsc

N = 10000
E = 320000
D_IN = 128
D_OUT = 40
D_OUT_PAD = 48

NC = 2
NS = 16
NW = NC * NS
CH = 125
NCHUNK = 80
EPW = CH * NCHUNK
N_TAB = 10240
ZROWS = N_TAB // NS

_MESH = dict(core_axis_name="c", subcore_axis_name="s")
_NBUF = 4


def _edge_pipeline(tab, src_v, dst_v, acc, bufs, gsems, ssems, nchunk):
  for k in range(_NBUF - 1):
    pltpu.async_copy(tab.at[src_v.at[k]], bufs[k], gsems[k])

  def body(i, _):
    for k in range(_NBUF):
      j = _NBUF * i + k
      kn = (k + _NBUF - 1) % _NBUF
      pltpu.make_async_copy(tab.at[src_v.at[j]], bufs[k], gsems[k]).wait()
      pltpu.async_copy(bufs[k], acc.at[dst_v.at[j]], ssems[k], add=True)

      @pl.when(j >= 1)
      def _():
        pltpu.make_async_copy(bufs[kn], acc.at[dst_v.at[j - 1]],
                              ssems[kn]).wait()

      @pl.when(j + _NBUF - 1 < nchunk)
      def _():
        pltpu.async_copy(tab.at[src_v.at[j + _NBUF - 1]], bufs[kn], gsems[kn])

    return 0

  lax.fori_loop(0, nchunk // _NBUF, body, 0)
  pltpu.make_async_copy(bufs[_NBUF - 1], acc.at[dst_v.at[nchunk - 1]],
                        ssems[_NBUF - 1]).wait()


def _make_agg(d):
  @functools.partial(
      pl.kernel,
      out_type=jax.ShapeDtypeStruct((NC, N_TAB, d), jnp.float32),
      mesh=plsc.VectorSubcoreMesh(**_MESH),
      compiler_params=pltpu.CompilerParams(use_tc_tiling_on_sc=False),
      scratch_types=(
          [pltpu.VMEM((NCHUNK, CH), jnp.int32)] * 2
          + [pltpu.VMEM((CH, d), jnp.float32)] * _NBUF
          + [pltpu.SemaphoreType.DMA] * (2 * _NBUF)
          + [pltpu.VMEM_SHARED((N_TAB, d), jnp.float32)]
      ),
  )
  def agg(tab_hbm, srcr_hbm, dstr_hbm, zer_hbm, out_hbm,
          src_v, dst_v, *rest):
    bufs, sems, acc = rest[:_NBUF], rest[_NBUF:3 * _NBUF], rest[-1]
    gsems, ssems = sems[:_NBUF], sems[_NBUF:]
    cid = lax.axis_index("c")
    sid = lax.axis_index("s")
    pltpu.sync_copy(srcr_hbm.at[sid, pl.ds(cid * NCHUNK, NCHUNK)], src_v)
    pltpu.sync_copy(dstr_hbm.at[sid, pl.ds(cid * NCHUNK, NCHUNK)], dst_v)
    pltpu.sync_copy(zer_hbm, acc.at[pl.ds(sid * ZROWS, ZROWS)])
    plsc.subcore_barrier()
    _edge_pipeline(tab_hbm, src_v, dst_v, acc, bufs, gsems, ssems, NCHUNK)
    plsc.subcore_barrier()
    pltpu.sync_copy(acc.at[pl.ds(sid * ZROWS, ZROWS)],
                    out_hbm.at[cid, pl.ds(sid * ZROWS, ZROWS)])

  return agg


NCHUNK2 = NCHUNK * NC
DH = D_IN // 2


@functools.partial(
    pl.kernel,
    out_type=jax.ShapeDtypeStruct((NC, N_TAB, DH), jnp.float32),
    mesh=plsc.VectorSubcoreMesh(**_MESH),
    compiler_params=pltpu.CompilerParams(use_tc_tiling_on_sc=False),
    scratch_types=(
        [pltpu.VMEM((NCHUNK2, CH), jnp.int32)] * 2
        + [pltpu.VMEM((CH, DH), jnp.float32)] * _NBUF
        + [pltpu.SemaphoreType.DMA] * (2 * _NBUF)
        + [pltpu.VMEM_SHARED((N_TAB, DH), jnp.float32)]
    ),
)
def _agg_split(tab_hbm, srcr_hbm, dstr_hbm, zer_hbm, out_hbm,
               src_v, dst_v, *rest):
  bufs, sems, acc = rest[:_NBUF], rest[_NBUF:3 * _NBUF], rest[-1]
  gsems, ssems = sems[:_NBUF], sems[_NBUF:]
  cid = lax.axis_index("c")
  sid = lax.axis_index("s")
  tab = tab_hbm.at[cid]
  pltpu.sync_copy(srcr_hbm.at[sid], src_v)
  pltpu.sync_copy(dstr_hbm.at[sid], dst_v)
  pltpu.sync_copy(zer_hbm, acc.at[pl.ds(sid * ZROWS, ZROWS)])
  plsc.subcore_barrier()
  _edge_pipeline(tab, src_v, dst_v, acc, bufs, gsems, ssems, NCHUNK2)
  plsc.subcore_barrier()
  pltpu.sync_copy(acc.at[pl.ds(sid * ZROWS, ZROWS)],
                  out_hbm.at[cid, pl.ds(sid * ZROWS, ZROWS)])


@functools.partial(
    pl.kernel,
    out_type=jax.ShapeDtypeStruct((NC, N_TAB, 16), jnp.float32),
    mesh=plsc.VectorSubcoreMesh(**_MESH),
    compiler_params=pltpu.CompilerParams(use_tc_tiling_on_sc=False),
    scratch_types=[
        pltpu.VMEM((NCHUNK, CH), jnp.int32),
        pltpu.VMEM((CH, 16), jnp.float32),
        pltpu.VMEM_SHARED((N_TAB, 16), jnp.float32),
    ],
)
def _deg(dstr_hbm, ones_hbm, zer_hbm, out_hbm, dst_v, ones_v, acc):
  cid = lax.axis_index("c")
  sid = lax.axis_index("s")
  pltpu.sync_copy(dstr_hbm.at[sid, pl.ds(cid * NCHUNK, NCHUNK)], dst_v)
  pltpu.sync_copy(ones_hbm, ones_v)
  pltpu.sync_copy(zer_hbm, acc.at[pl.ds(sid * ZROWS, ZROWS)])
  plsc.subcore_barrier()

  def body(j, _):
    pltpu.sync_copy(ones_v, acc.at[dst_v.at[j]], add=True)
    return 0

  lax.fori_loop(0, NCHUNK, body, 0)
  plsc.subcore_barrier()
  pltpu.sync_copy(acc.at[pl.ds(sid * ZROWS, ZROWS)],
                  out_hbm.at[cid, pl.ds(sid * ZROWS, ZROWS)])


_R = 1000


def _dinv_of(degp):
  deg = degp[0, :, 0] + degp[1, :, 0] + 1.0
  return lax.rsqrt(deg)


def _mm1_body(degp_ref, x_ref, w_ref, hs_ref):
  dinv = _dinv_of(degp_ref[...])
  h = jnp.dot(x_ref[...], w_ref[...], preferred_element_type=jnp.float32)
  hs = h * dinv[:, None]
  hs_ref[...] = jnp.stack([hs[:, :DH], hs[:, DH:]])


def _comb1_body(degp_ref, p_ref, hs_ref, b1_ref, w2_ref, gs_ref):
  dinv = _dinv_of(degp_ref[...])
  p = p_ref[...]
  hsp = hs_ref[...]
  agg = jnp.concatenate([p[0] + hsp[0], p[1] + hsp[1]], axis=1)
  s = agg * dinv[:, None] + b1_ref[...]
  h1 = jnp.maximum(s, 0.0)
  gs_ref[...] = jnp.dot(h1, w2_ref[...],
                        preferred_element_type=jnp.float32) * dinv[:, None]


def _final_body(degp_ref, q_ref, gs_ref, b2_ref, o_ref):
  dinv = _dinv_of(degp_ref[...])
  q = q_ref[...]
  z = (q[0] + q[1] + gs_ref[...]) * dinv[:, None] + b2_ref[...]
  z = z[:, :D_OUT]
  m = jnp.max(z, axis=1, keepdims=True)
  lse = jnp.log(jnp.sum(jnp.exp(z - m), axis=1, keepdims=True)) + m
  o_ref[...] = z - lse


def _degp_spec():
  return pl.BlockSpec((NC, _R, 16), lambda i: (0, i, 0))


_mm1 = pl.pallas_call(
    _mm1_body,
    grid=(N // _R,),
    in_specs=[
        _degp_spec(),
        pl.BlockSpec((_R, D_IN), lambda i: (i, 0)),
        pl.BlockSpec((D_IN, D_IN), lambda i: (0, 0)),
    ],
    out_specs=pl.BlockSpec((NC, _R, DH), lambda i: (0, i, 0)),
    out_shape=jax.ShapeDtypeStruct((NC, N, DH), jnp.float32),
)

_comb1 = pl.pallas_call(
    _comb1_body,
    grid=(N // _R,),
    in_specs=[
        _degp_spec(),
        pl.BlockSpec((NC, _R, DH), lambda i: (0, i, 0)),
        pl.BlockSpec((NC, _R, DH), lambda i: (0, i, 0)),
        pl.BlockSpec((1, D_IN), lambda i: (0, 0)),
        pl.BlockSpec((D_IN, D_OUT_PAD), lambda i: (0, 0)),
    ],
    out_specs=pl.BlockSpec((_R, D_OUT_PAD), lambda i: (i, 0)),
    out_shape=jax.ShapeDtypeStruct((N, D_OUT_PAD), jnp.float32),
)

_final = pl.pallas_call(
    _final_body,
    grid=(N // _R,),
    in_specs=[
        _degp_spec(),
        pl.BlockSpec((NC, _R, D_OUT_PAD), lambda i: (0, i, 0)),
        pl.BlockSpec((_R, D_OUT_PAD), lambda i: (i, 0)),
        pl.BlockSpec((1, D_OUT_PAD), lambda i: (0, 0)),
    ],
    out_specs=pl.BlockSpec((_R, D_OUT), lambda i: (i, 0)),
    out_shape=jax.ShapeDtypeStruct((N, D_OUT), jnp.float32),
)

_agg48 = _make_agg(D_OUT_PAD)


def kernel(x, edge, W1, b1, W2, b2):
  srcr2 = edge[0].reshape(NS, NCHUNK2, CH)
  dstr2 = edge[1].reshape(NS, NCHUNK2, CH)
  zer64 = jnp.zeros((ZROWS, DH), jnp.float32)
  zer48 = jnp.zeros((ZROWS, D_OUT_PAD), jnp.float32)
  zer16 = jnp.zeros((ZROWS, 16), jnp.float32)
  ones16 = jnp.ones((CH, 16), jnp.float32)
  w2p = jnp.pad(W2, ((0, 0), (0, D_OUT_PAD - D_OUT)))
  b1r = b1.reshape(1, D_IN)
  b2r = jnp.pad(b2, (0, D_OUT_PAD - D_OUT)).reshape(1, D_OUT_PAD)

  degp = _deg(dstr2, ones16, zer16)
  hsp = _mm1(degp, x, W1)
  p1 = _agg_split(hsp, srcr2, dstr2, zer64)
  gs = _comb1(degp, p1, hsp, b1r, w2p)
  p2 = _agg48(gs, srcr2, dstr2, zer48)
  return _final(degp, p2, gs, b2r)


# speedup vs baseline: 36.6469x; 1.0349x over previous
import functools

import jax
import jax.numpy as jnp
from jax import lax
from jax.experimental import pallas as pl
from jax.experimental.pallas import tpu as pltpu
from jax.experimental.pallas import tpu_sc as plsc

N = 10000
E = 320000
D_IN = 128
D_OUT = 40
D_OUT_PAD = 48

NC = 2
NS = 16
NW = NC * NS
CH = 125
NCHUNK = 80
EPW = CH * NCHUNK
N_TAB = 10240
ZROWS = N_TAB // NS

_MESH = dict(core_axis_name="c", subcore_axis_name="s")
_NBUF = 4


def _edge_pipeline(tab, src_v, dst_v, acc, bufs, gsems, ssems, nchunk):
  for k in range(_NBUF - 1):
    pltpu.async_copy(tab.at[src_v.at[k]], bufs[k], gsems[k])

  def body(i, _):
    for k in range(_NBUF):
      j = _NBUF * i + k
      kn = (k + _NBUF - 1) % _NBUF
      pltpu.make_async_copy(tab.at[src_v.at[j]], bufs[k], gsems[k]).wait()
      pltpu.async_copy(bufs[k], acc.at[dst_v.at[j]], ssems[k], add=True)

      @pl.when(j >= 1)
      def _():
        pltpu.make_async_copy(bufs[kn], acc.at[dst_v.at[j - 1]],
                              ssems[kn]).wait()

      @pl.when(j + _NBUF - 1 < nchunk)
      def _():
        pltpu.async_copy(tab.at[src_v.at[j + _NBUF - 1]], bufs[kn], gsems[kn])

    return 0

  lax.fori_loop(0, nchunk // _NBUF, body, 0)
  pltpu.make_async_copy(bufs[_NBUF - 1], acc.at[dst_v.at[nchunk - 1]],
                        ssems[_NBUF - 1]).wait()


def _make_agg(d):
  @functools.partial(
      pl.kernel,
      out_type=jax.ShapeDtypeStruct((NC, N_TAB, d), jnp.float32),
      mesh=plsc.VectorSubcoreMesh(**_MESH),
      compiler_params=pltpu.CompilerParams(use_tc_tiling_on_sc=False),
      scratch_types=(
          [pltpu.VMEM((NCHUNK, CH), jnp.int32)] * 2
          + [pltpu.VMEM((CH, d), jnp.float32)] * _NBUF
          + [pltpu.SemaphoreType.DMA] * (2 * _NBUF)
          + [pltpu.VMEM_SHARED((N_TAB, d), jnp.float32)]
      ),
  )
  def agg(tab_hbm, srcr_hbm, dstr_hbm, zer_hbm, out_hbm,
          src_v, dst_v, *rest):
    bufs, sems, acc = rest[:_NBUF], rest[_NBUF:3 * _NBUF], rest[-1]
    gsems, ssems = sems[:_NBUF], sems[_NBUF:]
    cid = lax.axis_index("c")
    sid = lax.axis_index("s")
    pltpu.sync_copy(srcr_hbm.at[sid, pl.ds(cid * NCHUNK, NCHUNK)], src_v)
    pltpu.sync_copy(dstr_hbm.at[sid, pl.ds(cid * NCHUNK, NCHUNK)], dst_v)
    pltpu.sync_copy(zer_hbm, acc.at[pl.ds(sid * ZROWS, ZROWS)])
    plsc.subcore_barrier()
    _edge_pipeline(tab_hbm, src_v, dst_v, acc, bufs, gsems, ssems, NCHUNK)
    plsc.subcore_barrier()
    pltpu.sync_copy(acc.at[pl.ds(sid * ZROWS, ZROWS)],
                    out_hbm.at[cid, pl.ds(sid * ZROWS, ZROWS)])

  return agg


NCHUNK2 = NCHUNK * NC
DH = D_IN // 2


@functools.partial(
    pl.kernel,
    out_type=jax.ShapeDtypeStruct((NC, N_TAB, DH), jnp.float32),
    mesh=plsc.VectorSubcoreMesh(**_MESH),
    compiler_params=pltpu.CompilerParams(use_tc_tiling_on_sc=False),
    scratch_types=(
        [pltpu.VMEM((NCHUNK2, CH), jnp.int32)] * 2
        + [pltpu.VMEM((CH, DH), jnp.float32)] * _NBUF
        + [pltpu.SemaphoreType.DMA] * (2 * _NBUF)
        + [pltpu.VMEM_SHARED((N_TAB, DH), jnp.float32)]
    ),
)
def _agg_split(tab_hbm, srcr_hbm, dstr_hbm, zer_hbm, out_hbm,
               src_v, dst_v, *rest):
  bufs, sems, acc = rest[:_NBUF], rest[_NBUF:3 * _NBUF], rest[-1]
  gsems, ssems = sems[:_NBUF], sems[_NBUF:]
  cid = lax.axis_index("c")
  sid = lax.axis_index("s")
  tab = tab_hbm.at[cid]
  pltpu.sync_copy(srcr_hbm.at[sid], src_v)
  pltpu.sync_copy(dstr_hbm.at[sid], dst_v)
  pltpu.sync_copy(zer_hbm, acc.at[pl.ds(sid * ZROWS, ZROWS)])
  plsc.subcore_barrier()
  _edge_pipeline(tab, src_v, dst_v, acc, bufs, gsems, ssems, NCHUNK2)
  plsc.subcore_barrier()
  pltpu.sync_copy(acc.at[pl.ds(sid * ZROWS, ZROWS)],
                  out_hbm.at[cid, pl.ds(sid * ZROWS, ZROWS)])


@functools.partial(
    pl.kernel,
    out_type=jax.ShapeDtypeStruct((NC, N_TAB, 16), jnp.float32),
    mesh=plsc.VectorSubcoreMesh(**_MESH),
    compiler_params=pltpu.CompilerParams(use_tc_tiling_on_sc=False),
    scratch_types=(
        [pltpu.VMEM((NCHUNK, CH), jnp.int32),
         pltpu.VMEM((CH, 16), jnp.float32)]
        + [pltpu.SemaphoreType.DMA] * _NBUF
        + [pltpu.VMEM_SHARED((N_TAB, 16), jnp.float32)]
    ),
)
def _deg(dstr_hbm, ones_hbm, zer_hbm, out_hbm, dst_v, ones_v, *rest):
  ssems, acc = rest[:_NBUF], rest[-1]
  cid = lax.axis_index("c")
  sid = lax.axis_index("s")
  pltpu.sync_copy(dstr_hbm.at[sid, pl.ds(cid * NCHUNK, NCHUNK)], dst_v)
  pltpu.sync_copy(ones_hbm, ones_v)
  pltpu.sync_copy(zer_hbm, acc.at[pl.ds(sid * ZROWS, ZROWS)])
  plsc.subcore_barrier()

  def body(i, _):
    for k in range(_NBUF):
      j = _NBUF * i + k
      pltpu.async_copy(ones_v, acc.at[dst_v.at[j]], ssems[k], add=True)

      @pl.when(j >= _NBUF - 1)
      def _():
        kp = (k + 1) % _NBUF
        pltpu.make_async_copy(ones_v, acc.at[dst_v.at[j - _NBUF + 1]],
                              ssems[kp]).wait()

    return 0

  lax.fori_loop(0, NCHUNK // _NBUF, body, 0)
  for k in range(_NBUF - 1):
    pltpu.make_async_copy(ones_v, acc.at[dst_v.at[NCHUNK - _NBUF + 1 + k]],
                          ssems[(k + 1) % _NBUF]).wait()
  plsc.subcore_barrier()
  pltpu.sync_copy(acc.at[pl.ds(sid * ZROWS, ZROWS)],
                  out_hbm.at[cid, pl.ds(sid * ZROWS, ZROWS)])


_R = 2000


def _dinv_of(degp):
  deg = degp[0, :, 0] + degp[1, :, 0] + 1.0
  return lax.rsqrt(deg)


def _mm1_body(degp_ref, x_ref, w_ref, hs_ref):
  dinv = _dinv_of(degp_ref[...])
  h = jnp.dot(x_ref[...], w_ref[...], preferred_element_type=jnp.float32)
  hs = h * dinv[:, None]
  hs_ref[...] = jnp.stack([hs[:, :DH], hs[:, DH:]])


def _comb1_body(degp_ref, p_ref, hs_ref, b1_ref, w2_ref, gs_ref):
  dinv = _dinv_of(degp_ref[...])
  p = p_ref[...]
  hsp = hs_ref[...]
  agg = jnp.concatenate([p[0] + hsp[0], p[1] + hsp[1]], axis=1)
  s = agg * dinv[:, None] + b1_ref[...]
  h1 = jnp.maximum(s, 0.0)
  gs_ref[...] = jnp.dot(h1, w2_ref[...],
                        preferred_element_type=jnp.float32) * dinv[:, None]


def _final_body(degp_ref, q_ref, gs_ref, b2_ref, o_ref):
  dinv = _dinv_of(degp_ref[...])
  q = q_ref[...]
  z = (q[0] + q[1] + gs_ref[...]) * dinv[:, None] + b2_ref[...]
  z = z[:, :D_OUT]
  m = jnp.max(z, axis=1, keepdims=True)
  lse = jnp.log(jnp.sum(jnp.exp(z - m), axis=1, keepdims=True)) + m
  o_ref[...] = z - lse


def _degp_spec():
  return pl.BlockSpec((NC, _R, 16), lambda i: (0, i, 0))


_mm1 = pl.pallas_call(
    _mm1_body,
    grid=(N // _R,),
    in_specs=[
        _degp_spec(),
        pl.BlockSpec((_R, D_IN), lambda i: (i, 0)),
        pl.BlockSpec((D_IN, D_IN), lambda i: (0, 0)),
    ],
    out_specs=pl.BlockSpec((NC, _R, DH), lambda i: (0, i, 0)),
    out_shape=jax.ShapeDtypeStruct((NC, N, DH), jnp.float32),
)

_comb1 = pl.pallas_call(
    _comb1_body,
    grid=(N // _R,),
    in_specs=[
        _degp_spec(),
        pl.BlockSpec((NC, _R, DH), lambda i: (0, i, 0)),
        pl.BlockSpec((NC, _R, DH), lambda i: (0, i, 0)),
        pl.BlockSpec((1, D_IN), lambda i: (0, 0)),
        pl.BlockSpec((D_IN, D_OUT_PAD), lambda i: (0, 0)),
    ],
    out_specs=pl.BlockSpec((_R, D_OUT_PAD), lambda i: (i, 0)),
    out_shape=jax.ShapeDtypeStruct((N, D_OUT_PAD), jnp.float32),
)

_final = pl.pallas_call(
    _final_body,
    grid=(N // _R,),
    in_specs=[
        _degp_spec(),
        pl.BlockSpec((NC, _R, D_OUT_PAD), lambda i: (0, i, 0)),
        pl.BlockSpec((_R, D_OUT_PAD), lambda i: (i, 0)),
        pl.BlockSpec((1, D_OUT_PAD), lambda i: (0, 0)),
    ],
    out_specs=pl.BlockSpec((_R, D_OUT), lambda i: (i, 0)),
    out_shape=jax.ShapeDtypeStruct((N, D_OUT), jnp.float32),
)

_agg48 = _make_agg(D_OUT_PAD)


def kernel(x, edge, W1, b1, W2, b2):
  srcr2 = edge[0].reshape(NS, NCHUNK2, CH)
  dstr2 = edge[1].reshape(NS, NCHUNK2, CH)
  zer64 = jnp.zeros((ZROWS, DH), jnp.float32)
  zer48 = jnp.zeros((ZROWS, D_OUT_PAD), jnp.float32)
  zer16 = jnp.zeros((ZROWS, 16), jnp.float32)
  ones16 = jnp.ones((CH, 16), jnp.float32)
  w2p = jnp.pad(W2, ((0, 0), (0, D_OUT_PAD - D_OUT)))
  b1r = b1.reshape(1, D_IN)
  b2r = jnp.pad(b2, (0, D_OUT_PAD - D_OUT)).reshape(1, D_OUT_PAD)

  degp = _deg(dstr2, ones16, zer16)
  hsp = _mm1(degp, x, W1)
  p1 = _agg_split(hsp, srcr2, dstr2, zer64)
  gs = _comb1(degp, p1, hsp, b1r, w2p)
  p2 = _agg48(gs, srcr2, dstr2, zer48)
  return _final(degp, p2, gs, b2r)


# speedup vs baseline: 38.0329x; 1.0378x over previous
import functools

import jax
import jax.numpy as jnp
from jax import lax
from jax.experimental import pallas as pl
from jax.experimental.pallas import tpu as pltpu
from jax.experimental.pallas import tpu_sc as plsc

N = 10000
E = 320000
D_IN = 128
D_OUT = 40
D_OUT_PAD = 48

NC = 2
NS = 16
NW = NC * NS
CH = 125
NCHUNK = 80
EPW = CH * NCHUNK
N_TAB = 10240
ZROWS = N_TAB // NS

_MESH = dict(core_axis_name="c", subcore_axis_name="s")
_NBUF = 5


def _edge_pipeline(tab, src_v, dst_v, acc, bufs, gsems, ssems, nchunk):
  for k in range(_NBUF - 1):
    pltpu.async_copy(tab.at[src_v.at[k]], bufs[k], gsems[k])

  def body(i, _):
    for k in range(_NBUF):
      j = _NBUF * i + k
      kn = (k + _NBUF - 1) % _NBUF
      pltpu.make_async_copy(tab.at[src_v.at[j]], bufs[k], gsems[k]).wait()
      pltpu.async_copy(bufs[k], acc.at[dst_v.at[j]], ssems[k], add=True)

      @pl.when(j >= 1)
      def _():
        pltpu.make_async_copy(bufs[kn], acc.at[dst_v.at[j - 1]],
                              ssems[kn]).wait()

      @pl.when(j + _NBUF - 1 < nchunk)
      def _():
        pltpu.async_copy(tab.at[src_v.at[j + _NBUF - 1]], bufs[kn], gsems[kn])

    return 0

  lax.fori_loop(0, nchunk // _NBUF, body, 0)
  pltpu.make_async_copy(bufs[_NBUF - 1], acc.at[dst_v.at[nchunk - 1]],
                        ssems[_NBUF - 1]).wait()


def _make_agg(d):
  @functools.partial(
      pl.kernel,
      out_type=jax.ShapeDtypeStruct((NC, N_TAB, d), jnp.float32),
      mesh=plsc.VectorSubcoreMesh(**_MESH),
      compiler_params=pltpu.CompilerParams(use_tc_tiling_on_sc=False),
      scratch_types=(
          [pltpu.VMEM((NCHUNK, CH), jnp.int32)] * 2
          + [pltpu.VMEM((CH, d), jnp.float32)] * _NBUF
          + [pltpu.SemaphoreType.DMA] * (2 * _NBUF)
          + [pltpu.VMEM_SHARED((N_TAB, d), jnp.float32)]
      ),
  )
  def agg(tab_hbm, srcr_hbm, dstr_hbm, zer_hbm, out_hbm,
          src_v, dst_v, *rest):
    bufs, sems, acc = rest[:_NBUF], rest[_NBUF:3 * _NBUF], rest[-1]
    gsems, ssems = sems[:_NBUF], sems[_NBUF:]
    cid = lax.axis_index("c")
    sid = lax.axis_index("s")
    pltpu.sync_copy(srcr_hbm.at[sid, pl.ds(cid * NCHUNK, NCHUNK)], src_v)
    pltpu.sync_copy(dstr_hbm.at[sid, pl.ds(cid * NCHUNK, NCHUNK)], dst_v)
    pltpu.sync_copy(zer_hbm, acc.at[pl.ds(sid * ZROWS, ZROWS)])
    plsc.subcore_barrier()
    _edge_pipeline(tab_hbm, src_v, dst_v, acc, bufs, gsems, ssems, NCHUNK)
    plsc.subcore_barrier()
    pltpu.sync_copy(acc.at[pl.ds(sid * ZROWS, ZROWS)],
                    out_hbm.at[cid, pl.ds(sid * ZROWS, ZROWS)])

  return agg


NCHUNK2 = NCHUNK * NC
DH = D_IN // 2


@functools.partial(
    pl.kernel,
    out_type=jax.ShapeDtypeStruct((NC, N_TAB, DH), jnp.float32),
    mesh=plsc.VectorSubcoreMesh(**_MESH),
    compiler_params=pltpu.CompilerParams(use_tc_tiling_on_sc=False),
    scratch_types=(
        [pltpu.VMEM((NCHUNK2, CH), jnp.int32)] * 2
        + [pltpu.VMEM((CH, DH), jnp.float32)] * _NBUF
        + [pltpu.SemaphoreType.DMA] * (2 * _NBUF)
        + [pltpu.VMEM_SHARED((N_TAB, DH), jnp.float32)]
    ),
)
def _agg_split(tab_hbm, srcr_hbm, dstr_hbm, zer_hbm, out_hbm,
               src_v, dst_v, *rest):
  bufs, sems, acc = rest[:_NBUF], rest[_NBUF:3 * _NBUF], rest[-1]
  gsems, ssems = sems[:_NBUF], sems[_NBUF:]
  cid = lax.axis_index("c")
  sid = lax.axis_index("s")
  tab = tab_hbm.at[cid]
  pltpu.sync_copy(srcr_hbm.at[sid], src_v)
  pltpu.sync_copy(dstr_hbm.at[sid], dst_v)
  pltpu.sync_copy(zer_hbm, acc.at[pl.ds(sid * ZROWS, ZROWS)])
  plsc.subcore_barrier()
  _edge_pipeline(tab, src_v, dst_v, acc, bufs, gsems, ssems, NCHUNK2)
  plsc.subcore_barrier()
  pltpu.sync_copy(acc.at[pl.ds(sid * ZROWS, ZROWS)],
                  out_hbm.at[cid, pl.ds(sid * ZROWS, ZROWS)])


@functools.partial(
    pl.kernel,
    out_type=jax.ShapeDtypeStruct((NC, N_TAB, 16), jnp.float32),
    mesh=plsc.VectorSubcoreMesh(**_MESH),
    compiler_params=pltpu.CompilerParams(use_tc_tiling_on_sc=False),
    scratch_types=(
        [pltpu.VMEM((NCHUNK, CH), jnp.int32),
         pltpu.VMEM((CH, 16), jnp.float32)]
        + [pltpu.SemaphoreType.DMA] * _NBUF
        + [pltpu.VMEM_SHARED((N_TAB, 16), jnp.float32)]
    ),
)
def _deg(dstr_hbm, ones_hbm, zer_hbm, out_hbm, dst_v, ones_v, *rest):
  ssems, acc = rest[:_NBUF], rest[-1]
  cid = lax.axis_index("c")
  sid = lax.axis_index("s")
  pltpu.sync_copy(dstr_hbm.at[sid, pl.ds(cid * NCHUNK, NCHUNK)], dst_v)
  pltpu.sync_copy(ones_hbm, ones_v)
  pltpu.sync_copy(zer_hbm, acc.at[pl.ds(sid * ZROWS, ZROWS)])
  plsc.subcore_barrier()

  def body(i, _):
    for k in range(_NBUF):
      j = _NBUF * i + k
      pltpu.async_copy(ones_v, acc.at[dst_v.at[j]], ssems[k], add=True)

      @pl.when(j >= _NBUF - 1)
      def _():
        kp = (k + 1) % _NBUF
        pltpu.make_async_copy(ones_v, acc.at[dst_v.at[j - _NBUF + 1]],
                              ssems[kp]).wait()

    return 0

  lax.fori_loop(0, NCHUNK // _NBUF, body, 0)
  for k in range(_NBUF - 1):
    pltpu.make_async_copy(ones_v, acc.at[dst_v.at[NCHUNK - _NBUF + 1 + k]],
                          ssems[(k + 1) % _NBUF]).wait()
  plsc.subcore_barrier()
  pltpu.sync_copy(acc.at[pl.ds(sid * ZROWS, ZROWS)],
                  out_hbm.at[cid, pl.ds(sid * ZROWS, ZROWS)])


_R = 2000


def _dinv_of(degp):
  deg = degp[0, :, 0] + degp[1, :, 0] + 1.0
  return lax.rsqrt(deg)


def _mm1_body(degp_ref, x_ref, w_ref, hs_ref):
  dinv = _dinv_of(degp_ref[...])
  h = jnp.dot(x_ref[...], w_ref[...], preferred_element_type=jnp.float32)
  hs = h * dinv[:, None]
  hs_ref[...] = jnp.stack([hs[:, :DH], hs[:, DH:]])


def _comb1_body(degp_ref, p_ref, hs_ref, b1_ref, w2_ref, gs_ref):
  dinv = _dinv_of(degp_ref[...])
  p = p_ref[...]
  hsp = hs_ref[...]
  agg = jnp.concatenate([p[0] + hsp[0], p[1] + hsp[1]], axis=1)
  s = agg * dinv[:, None] + b1_ref[...]
  h1 = jnp.maximum(s, 0.0)
  gs_ref[...] = jnp.dot(h1, w2_ref[...],
                        preferred_element_type=jnp.float32) * dinv[:, None]


def _final_body(degp_ref, q_ref, gs_ref, b2_ref, o_ref):
  dinv = _dinv_of(degp_ref[...])
  q = q_ref[...]
  z = (q[0] + q[1] + gs_ref[...]) * dinv[:, None] + b2_ref[...]
  z = z[:, :D_OUT]
  m = jnp.max(z, axis=1, keepdims=True)
  lse = jnp.log(jnp.sum(jnp.exp(z - m), axis=1, keepdims=True)) + m
  o_ref[...] = z - lse


def _degp_spec():
  return pl.BlockSpec((NC, _R, 16), lambda i: (0, i, 0))


_mm1 = pl.pallas_call(
    _mm1_body,
    grid=(N // _R,),
    in_specs=[
        _degp_spec(),
        pl.BlockSpec((_R, D_IN), lambda i: (i, 0)),
        pl.BlockSpec((D_IN, D_IN), lambda i: (0, 0)),
    ],
    out_specs=pl.BlockSpec((NC, _R, DH), lambda i: (0, i, 0)),
    out_shape=jax.ShapeDtypeStruct((NC, N, DH), jnp.float32),
)

_comb1 = pl.pallas_call(
    _comb1_body,
    grid=(N // _R,),
    in_specs=[
        _degp_spec(),
        pl.BlockSpec((NC, _R, DH), lambda i: (0, i, 0)),
        pl.BlockSpec((NC, _R, DH), lambda i: (0, i, 0)),
        pl.BlockSpec((1, D_IN), lambda i: (0, 0)),
        pl.BlockSpec((D_IN, D_OUT_PAD), lambda i: (0, 0)),
    ],
    out_specs=pl.BlockSpec((_R, D_OUT_PAD), lambda i: (i, 0)),
    out_shape=jax.ShapeDtypeStruct((N, D_OUT_PAD), jnp.float32),
)

_final = pl.pallas_call(
    _final_body,
    grid=(N // _R,),
    in_specs=[
        _degp_spec(),
        pl.BlockSpec((NC, _R, D_OUT_PAD), lambda i: (0, i, 0)),
        pl.BlockSpec((_R, D_OUT_PAD), lambda i: (i, 0)),
        pl.BlockSpec((1, D_OUT_PAD), lambda i: (0, 0)),
    ],
    out_specs=pl.BlockSpec((_R, D_OUT), lambda i: (i, 0)),
    out_shape=jax.ShapeDtypeStruct((N, D_OUT), jnp.float32),
)

_agg48 = _make_agg(D_OUT_PAD)


def kernel(x, edge, W1, b1, W2, b2):
  srcr2 = edge[0].reshape(NS, NCHUNK2, CH)
  dstr2 = edge[1].reshape(NS, NCHUNK2, CH)
  zer64 = jnp.zeros((ZROWS, DH), jnp.float32)
  zer48 = jnp.zeros((ZROWS, D_OUT_PAD), jnp.float32)
  zer16 = jnp.zeros((ZROWS, 16), jnp.float32)
  ones16 = jnp.ones((CH, 16), jnp.float32)
  w2p = jnp.pad(W2, ((0, 0), (0, D_OUT_PAD - D_OUT)))
  b1r = b1.reshape(1, D_IN)
  b2r = jnp.pad(b2, (0, D_OUT_PAD - D_OUT)).reshape(1, D_OUT_PAD)

  degp = _deg(dstr2, ones16, zer16)
  hsp = _mm1(degp, x, W1)
  p1 = _agg_split(hsp, srcr2, dstr2, zer64)
  gs = _comb1(degp, p1, hsp, b1r, w2p)
  p2 = _agg48(gs, srcr2, dstr2, zer48)
  return _final(degp, p2, gs, b2r)


# speedup vs baseline: 38.5844x; 1.0145x over previous
import functools

import jax
import jax.numpy as jnp
from jax import lax
from jax.experimental import pallas as pl
from jax.experimental.pallas import tpu as pltpu
from jax.experimental.pallas import tpu_sc as plsc

N = 10000
E = 320000
D_IN = 128
D_OUT = 40
D_OUT_PAD = 48

NC = 2
NS = 16
NW = NC * NS
CH = 125
NCHUNK = 80
EPW = CH * NCHUNK
N_TAB = 10240
ZROWS = N_TAB // NS

_MESH = dict(core_axis_name="c", subcore_axis_name="s")
_NBUF = 5


def _edge_pipeline(tab, src_v, dst_v, acc, bufs, gsems, ssems, nchunk):
  for k in range(_NBUF - 1):
    pltpu.async_copy(tab.at[src_v.at[k]], bufs[k], gsems[k])

  def body(i, _):
    for k in range(_NBUF):
      j = _NBUF * i + k
      kn = (k + _NBUF - 1) % _NBUF
      pltpu.make_async_copy(tab.at[src_v.at[j]], bufs[k], gsems[k]).wait()
      pltpu.async_copy(bufs[k], acc.at[dst_v.at[j]], ssems[k], add=True)

      @pl.when(j >= 1)
      def _():
        pltpu.make_async_copy(bufs[kn], acc.at[dst_v.at[j - 1]],
                              ssems[kn]).wait()

      @pl.when(j + _NBUF - 1 < nchunk)
      def _():
        pltpu.async_copy(tab.at[src_v.at[j + _NBUF - 1]], bufs[kn], gsems[kn])

    return 0

  lax.fori_loop(0, nchunk // _NBUF, body, 0)
  pltpu.make_async_copy(bufs[_NBUF - 1], acc.at[dst_v.at[nchunk - 1]],
                        ssems[_NBUF - 1]).wait()


def _make_agg(d):
  @functools.partial(
      pl.kernel,
      out_type=jax.ShapeDtypeStruct((NC, N_TAB, d), jnp.float32),
      mesh=plsc.VectorSubcoreMesh(**_MESH),
      compiler_params=pltpu.CompilerParams(use_tc_tiling_on_sc=False),
      scratch_types=(
          [pltpu.VMEM((NCHUNK, CH), jnp.int32)] * 2
          + [pltpu.VMEM((CH, d), jnp.float32)] * _NBUF
          + [pltpu.SemaphoreType.DMA] * (2 * _NBUF)
          + [pltpu.VMEM_SHARED((N_TAB, d), jnp.float32)]
      ),
  )
  def agg(tab_hbm, srcr_hbm, dstr_hbm, zer_hbm, out_hbm,
          src_v, dst_v, *rest):
    bufs, sems, acc = rest[:_NBUF], rest[_NBUF:3 * _NBUF], rest[-1]
    gsems, ssems = sems[:_NBUF], sems[_NBUF:]
    cid = lax.axis_index("c")
    sid = lax.axis_index("s")
    pltpu.sync_copy(srcr_hbm.at[sid, pl.ds(cid * NCHUNK, NCHUNK)], src_v)
    pltpu.sync_copy(dstr_hbm.at[sid, pl.ds(cid * NCHUNK, NCHUNK)], dst_v)
    pltpu.sync_copy(zer_hbm, acc.at[pl.ds(sid * ZROWS, ZROWS)])
    plsc.subcore_barrier()
    _edge_pipeline(tab_hbm, src_v, dst_v, acc, bufs, gsems, ssems, NCHUNK)
    plsc.subcore_barrier()
    pltpu.sync_copy(acc.at[pl.ds(sid * ZROWS, ZROWS)],
                    out_hbm.at[cid, pl.ds(sid * ZROWS, ZROWS)])

  return agg


NCHUNK2 = NCHUNK * NC
DH = D_IN // 2


@functools.partial(
    pl.kernel,
    out_type=jax.ShapeDtypeStruct((NC, N_TAB, DH), jnp.float32),
    mesh=plsc.VectorSubcoreMesh(**_MESH),
    compiler_params=pltpu.CompilerParams(use_tc_tiling_on_sc=False),
    scratch_types=(
        [pltpu.VMEM((NCHUNK2, CH), jnp.int32)] * 2
        + [pltpu.VMEM((CH, DH), jnp.float32)] * _NBUF
        + [pltpu.SemaphoreType.DMA] * (2 * _NBUF)
        + [pltpu.VMEM_SHARED((N_TAB, DH), jnp.float32)]
    ),
)
def _agg_split(tab_hbm, srcr_hbm, dstr_hbm, zer_hbm, out_hbm,
               src_v, dst_v, *rest):
  bufs, sems, acc = rest[:_NBUF], rest[_NBUF:3 * _NBUF], rest[-1]
  gsems, ssems = sems[:_NBUF], sems[_NBUF:]
  cid = lax.axis_index("c")
  sid = lax.axis_index("s")
  tab = tab_hbm.at[cid]
  pltpu.sync_copy(srcr_hbm.at[sid], src_v)
  pltpu.sync_copy(dstr_hbm.at[sid], dst_v)
  pltpu.sync_copy(zer_hbm, acc.at[pl.ds(sid * ZROWS, ZROWS)])
  plsc.subcore_barrier()
  _edge_pipeline(tab, src_v, dst_v, acc, bufs, gsems, ssems, NCHUNK2)
  plsc.subcore_barrier()
  pltpu.sync_copy(acc.at[pl.ds(sid * ZROWS, ZROWS)],
                  out_hbm.at[cid, pl.ds(sid * ZROWS, ZROWS)])


@functools.partial(
    pl.kernel,
    out_type=jax.ShapeDtypeStruct((NC, N_TAB, 8), jnp.float32),
    mesh=plsc.VectorSubcoreMesh(**_MESH),
    compiler_params=pltpu.CompilerParams(use_tc_tiling_on_sc=False),
    scratch_types=(
        [pltpu.VMEM((NCHUNK, CH), jnp.int32),
         pltpu.VMEM((CH, 8), jnp.float32)]
        + [pltpu.SemaphoreType.DMA] * _NBUF
        + [pltpu.VMEM_SHARED((N_TAB, 8), jnp.float32)]
    ),
)
def _deg(dstr_hbm, ones_hbm, zer_hbm, out_hbm, dst_v, ones_v, *rest):
  ssems, acc = rest[:_NBUF], rest[-1]
  cid = lax.axis_index("c")
  sid = lax.axis_index("s")
  pltpu.sync_copy(dstr_hbm.at[sid, pl.ds(cid * NCHUNK, NCHUNK)], dst_v)
  pltpu.sync_copy(ones_hbm, ones_v)
  pltpu.sync_copy(zer_hbm, acc.at[pl.ds(sid * ZROWS, ZROWS)])
  plsc.subcore_barrier()

  def body(i, _):
    for k in range(_NBUF):
      j = _NBUF * i + k
      pltpu.async_copy(ones_v, acc.at[dst_v.at[j]], ssems[k], add=True)

      @pl.when(j >= _NBUF - 1)
      def _():
        kp = (k + 1) % _NBUF
        pltpu.make_async_copy(ones_v, acc.at[dst_v.at[j - _NBUF + 1]],
                              ssems[kp]).wait()

    return 0

  lax.fori_loop(0, NCHUNK // _NBUF, body, 0)
  for k in range(_NBUF - 1):
    pltpu.make_async_copy(ones_v, acc.at[dst_v.at[NCHUNK - _NBUF + 1 + k]],
                          ssems[(k + 1) % _NBUF]).wait()
  plsc.subcore_barrier()
  pltpu.sync_copy(acc.at[pl.ds(sid * ZROWS, ZROWS)],
                  out_hbm.at[cid, pl.ds(sid * ZROWS, ZROWS)])


_R = 2000


def _dinv_of(degp):
  deg = degp[0, :, 0] + degp[1, :, 0] + 1.0
  return lax.rsqrt(deg)


def _mm1_body(degp_ref, x_ref, w_ref, hs_ref):
  dinv = _dinv_of(degp_ref[...])
  h = jnp.dot(x_ref[...], w_ref[...], preferred_element_type=jnp.float32)
  hs = h * dinv[:, None]
  hs_ref[...] = jnp.stack([hs[:, :DH], hs[:, DH:]])


def _comb1_body(degp_ref, p_ref, hs_ref, b1_ref, w2_ref, gs_ref):
  dinv = _dinv_of(degp_ref[...])
  p = p_ref[...]
  hsp = hs_ref[...]
  agg = jnp.concatenate([p[0] + hsp[0], p[1] + hsp[1]], axis=1)
  s = agg * dinv[:, None] + b1_ref[...]
  h1 = jnp.maximum(s, 0.0)
  gs_ref[...] = jnp.dot(h1, w2_ref[...],
                        preferred_element_type=jnp.float32) * dinv[:, None]


def _final_body(degp_ref, q_ref, gs_ref, b2_ref, o_ref):
  dinv = _dinv_of(degp_ref[...])
  q = q_ref[...]
  z = (q[0] + q[1] + gs_ref[...]) * dinv[:, None] + b2_ref[...]
  z = z[:, :D_OUT]
  m = jnp.max(z, axis=1, keepdims=True)
  lse = jnp.log(jnp.sum(jnp.exp(z - m), axis=1, keepdims=True)) + m
  o_ref[...] = z - lse


def _degp_spec():
  return pl.BlockSpec((NC, _R, 8), lambda i: (0, i, 0))


_mm1 = pl.pallas_call(
    _mm1_body,
    grid=(N // _R,),
    in_specs=[
        _degp_spec(),
        pl.BlockSpec((_R, D_IN), lambda i: (i, 0)),
        pl.BlockSpec((D_IN, D_IN), lambda i: (0, 0)),
    ],
    out_specs=pl.BlockSpec((NC, _R, DH), lambda i: (0, i, 0)),
    out_shape=jax.ShapeDtypeStruct((NC, N, DH), jnp.float32),
)

_comb1 = pl.pallas_call(
    _comb1_body,
    grid=(N // _R,),
    in_specs=[
        _degp_spec(),
        pl.BlockSpec((NC, _R, DH), lambda i: (0, i, 0)),
        pl.BlockSpec((NC, _R, DH), lambda i: (0, i, 0)),
        pl.BlockSpec((1, D_IN), lambda i: (0, 0)),
        pl.BlockSpec((D_IN, D_OUT_PAD), lambda i: (0, 0)),
    ],
    out_specs=pl.BlockSpec((_R, D_OUT_PAD), lambda i: (i, 0)),
    out_shape=jax.ShapeDtypeStruct((N, D_OUT_PAD), jnp.float32),
)

_final = pl.pallas_call(
    _final_body,
    grid=(N // _R,),
    in_specs=[
        _degp_spec(),
        pl.BlockSpec((NC, _R, D_OUT_PAD), lambda i: (0, i, 0)),
        pl.BlockSpec((_R, D_OUT_PAD), lambda i: (i, 0)),
        pl.BlockSpec((1, D_OUT_PAD), lambda i: (0, 0)),
    ],
    out_specs=pl.BlockSpec((_R, D_OUT), lambda i: (i, 0)),
    out_shape=jax.ShapeDtypeStruct((N, D_OUT), jnp.float32),
)

_agg48 = _make_agg(D_OUT_PAD)


def kernel(x, edge, W1, b1, W2, b2):
  srcr2 = edge[0].reshape(NS, NCHUNK2, CH)
  dstr2 = edge[1].reshape(NS, NCHUNK2, CH)
  zer64 = jnp.zeros((ZROWS, DH), jnp.float32)
  zer48 = jnp.zeros((ZROWS, D_OUT_PAD), jnp.float32)
  zer16 = jnp.zeros((ZROWS, 8), jnp.float32)
  ones16 = jnp.ones((CH, 8), jnp.float32)
  w2p = jnp.pad(W2, ((0, 0), (0, D_OUT_PAD - D_OUT)))
  b1r = b1.reshape(1, D_IN)
  b2r = jnp.pad(b2, (0, D_OUT_PAD - D_OUT)).reshape(1, D_OUT_PAD)

  degp = _deg(dstr2, ones16, zer16)
  hsp = _mm1(degp, x, W1)
  p1 = _agg_split(hsp, srcr2, dstr2, zer64)
  gs = _comb1(degp, p1, hsp, b1r, w2p)
  p2 = _agg48(gs, srcr2, dstr2, zer48)
  return _final(degp, p2, gs, b2r)
